# baseline (speedup 1.0000x reference)
import functools

import jax
import jax.numpy as jnp
from jax import lax
from jax.experimental import pallas as pl
from jax.experimental.pallas import tpu as pltpu
from jax.experimental.pallas import tpu_sc as plsc


_SC_CHUNKS = 4


def _sc_gather(table, idx_flat):
    info = plsc.get_sparse_core_info()
    num_workers = info.num_cores * info.num_subcores
    b = idx_flat.shape[0]
    d = table.shape[1]
    b_per_w = b // num_workers
    rows_per_chunk = b_per_w // _SC_CHUNKS
    mesh = plsc.VectorSubcoreMesh(core_axis_name="c", subcore_axis_name="s")

    @functools.partial(
        pl.kernel,
        mesh=mesh,
        out_type=jax.ShapeDtypeStruct((b, d), jnp.float32),
        scratch_types=[
            pltpu.VMEM((b_per_w,), jnp.int32),
        ]
        + [pltpu.VMEM((rows_per_chunk, d), jnp.float32)] * _SC_CHUNKS
        + [pltpu.SemaphoreType.DMA] * (2 * _SC_CHUNKS),
    )
    def gather_k(table_hbm, idx_hbm, out_hbm, idx_v, *bufs_and_sems):
        bufs = bufs_and_sems[:_SC_CHUNKS]
        gsems = bufs_and_sems[_SC_CHUNKS : 2 * _SC_CHUNKS]
        osems = bufs_and_sems[2 * _SC_CHUNKS :]
        wid = lax.axis_index("s") * info.num_cores + lax.axis_index("c")
        base = wid * b_per_w
        pltpu.sync_copy(idx_hbm.at[pl.ds(base, b_per_w)], idx_v)
        gathers = [
            pltpu.async_copy(
                table_hbm.at[idx_v.at[pl.ds(c * rows_per_chunk, rows_per_chunk)]],
                bufs[c],
                gsems[c],
            )
            for c in range(_SC_CHUNKS)
        ]
        scatters = []
        for c in range(_SC_CHUNKS):
            gathers[c].wait()
            scatters.append(
                pltpu.async_copy(
                    bufs[c],
                    out_hbm.at[pl.ds(base + c * rows_per_chunk, rows_per_chunk)],
                    osems[c],
                )
            )
        for s in scatters:
            s.wait()

    return gather_k(table, idx_flat)


def _tc_body(tok_ref, vc_ref, out_ref):
    t = tok_ref[0]
    vc = vc_ref[0]
    k = vc.shape[0]
    sims = lax.dot_general(
        vc, t, (((1,), (1,)), ((), ())), preferred_element_type=jnp.float32
    )
    s2 = jnp.concatenate([sims, -sims], axis=0)
    m = jnp.max(s2, axis=-1, keepdims=True)
    e = jnp.exp(s2 - m)
    p = e / jnp.sum(e, axis=-1, keepdims=True)
    w = lax.dot_general(
        p, t, (((1,), (0,)), ((), ())), preferred_element_type=jnp.float32
    )
    nrm = jnp.sqrt(jnp.sum(w * w, axis=-1, keepdims=True))
    w = w / jnp.maximum(nrm, 1e-12)
    out_ref[0, 0] = w[:k]
    out_ref[1, 0] = w[k:]


def kernel(tokens, topk_indices, visual_concepts):
    b, _, d = tokens.shape
    k = topk_indices.shape[1]
    n = tokens.shape[1] - 1
    idx_flat = topk_indices.reshape(-1).astype(jnp.int32)
    vc = _sc_gather(visual_concepts, idx_flat).reshape(b, k, d)
    t = lax.slice(tokens, (0, 1, 0), (b, n + 1, d))
    out = pl.pallas_call(
        _tc_body,
        grid=(b,),
        in_specs=[
            pl.BlockSpec((1, n, d), lambda i: (i, 0, 0)),
            pl.BlockSpec((1, k, d), lambda i: (i, 0, 0)),
        ],
        out_specs=pl.BlockSpec((2, 1, k, d), lambda i: (0, i, 0, 0)),
        out_shape=jax.ShapeDtypeStruct((2, b, k, d), jnp.float32),
        compiler_params=pltpu.CompilerParams(
            dimension_semantics=("parallel",),
        ),
    )(t, vc)
    return out


# speedup vs baseline: 1.2226x; 1.2226x over previous
import functools

import jax
import jax.numpy as jnp
from jax import lax
from jax.experimental import pallas as pl
from jax.experimental.pallas import tpu as pltpu
from jax.experimental.pallas import tpu_sc as plsc


_SC_CHUNKS = 4


def _sc_gather(table, idx_flat):
    info = plsc.get_sparse_core_info()
    num_workers = info.num_cores * info.num_subcores
    b = idx_flat.shape[0]
    d = table.shape[1]
    b_per_w = b // num_workers
    rows_per_chunk = b_per_w // _SC_CHUNKS
    mesh = plsc.VectorSubcoreMesh(core_axis_name="c", subcore_axis_name="s")

    @functools.partial(
        pl.kernel,
        mesh=mesh,
        out_type=jax.ShapeDtypeStruct((b, d), jnp.float32),
        scratch_types=[
            pltpu.VMEM((b_per_w,), jnp.int32),
        ]
        + [pltpu.VMEM((rows_per_chunk, d), jnp.float32)] * _SC_CHUNKS
        + [pltpu.SemaphoreType.DMA] * (2 * _SC_CHUNKS),
    )
    def gather_k(table_hbm, idx_hbm, out_hbm, idx_v, *bufs_and_sems):
        bufs = bufs_and_sems[:_SC_CHUNKS]
        gsems = bufs_and_sems[_SC_CHUNKS : 2 * _SC_CHUNKS]
        osems = bufs_and_sems[2 * _SC_CHUNKS :]
        wid = lax.axis_index("s") * info.num_cores + lax.axis_index("c")
        base = wid * b_per_w
        pltpu.sync_copy(idx_hbm.at[pl.ds(base, b_per_w)], idx_v)
        gathers = [
            pltpu.async_copy(
                table_hbm.at[idx_v.at[pl.ds(c * rows_per_chunk, rows_per_chunk)]],
                bufs[c],
                gsems[c],
            )
            for c in range(_SC_CHUNKS)
        ]
        scatters = []
        for c in range(_SC_CHUNKS):
            gathers[c].wait()
            scatters.append(
                pltpu.async_copy(
                    bufs[c],
                    out_hbm.at[pl.ds(base + c * rows_per_chunk, rows_per_chunk)],
                    osems[c],
                )
            )
        for s in scatters:
            s.wait()

    return gather_k(table, idx_flat)


def _tc_body(tok_hbm, vc_ref, out_ref, tbufs, sems):
    i = pl.program_id(0)
    nb = pl.num_programs(0)

    def _start(j):
        pltpu.make_async_copy(tok_hbm.at[j], tbufs.at[j % 2], sems.at[j % 2]).start()

    @pl.when(i == 0)
    def _():
        _start(0)

    @pl.when(i + 1 < nb)
    def _():
        _start(i + 1)

    pltpu.make_async_copy(tok_hbm.at[i], tbufs.at[i % 2], sems.at[i % 2]).wait()
    t = tbufs[i % 2]
    vc = vc_ref[0]
    k = vc.shape[0]
    sims = lax.dot_general(
        vc, t, (((1,), (1,)), ((), ())), preferred_element_type=jnp.float32
    )
    s2 = jnp.concatenate([sims, -sims], axis=0)
    col = lax.broadcasted_iota(jnp.int32, s2.shape, 1)
    s2 = jnp.where(col == 0, -jnp.inf, s2)
    m = jnp.max(s2, axis=-1, keepdims=True)
    e = jnp.exp(s2 - m)
    p = e / jnp.sum(e, axis=-1, keepdims=True)
    w = lax.dot_general(
        p, t, (((1,), (0,)), ((), ())), preferred_element_type=jnp.float32
    )
    nrm = jnp.sqrt(jnp.sum(w * w, axis=-1, keepdims=True))
    w = w / jnp.maximum(nrm, 1e-12)
    out_ref[0, 0] = w[:k]
    out_ref[1, 0] = w[k:]


def kernel(tokens, topk_indices, visual_concepts):
    b, n1, d = tokens.shape
    k = topk_indices.shape[1]
    idx_flat = topk_indices.reshape(-1).astype(jnp.int32)
    vc = _sc_gather(visual_concepts, idx_flat).reshape(b, k, d)
    out = pl.pallas_call(
        _tc_body,
        grid=(b,),
        in_specs=[
            pl.BlockSpec(memory_space=pl.ANY),
            pl.BlockSpec((1, k, d), lambda i: (i, 0, 0)),
        ],
        out_specs=pl.BlockSpec((2, 1, k, d), lambda i: (0, i, 0, 0)),
        out_shape=jax.ShapeDtypeStruct((2, b, k, d), jnp.float32),
        scratch_shapes=[
            pltpu.VMEM((2, n1, d), jnp.float32),
            pltpu.SemaphoreType.DMA((2,)),
        ],
        compiler_params=pltpu.CompilerParams(
            dimension_semantics=("arbitrary",),
        ),
    )(tokens, vc)
    return out


# speedup vs baseline: 1.4388x; 1.1768x over previous
import functools

import jax
import jax.numpy as jnp
from jax import lax
from jax.experimental import pallas as pl
from jax.experimental.pallas import tpu as pltpu
from jax.experimental.pallas import tpu_sc as plsc


_SC_CHUNKS = 4


def _sc_gather(table, idx_flat):
    info = plsc.get_sparse_core_info()
    num_workers = info.num_cores * info.num_subcores
    b = idx_flat.shape[0]
    d = table.shape[1]
    b_per_w = b // num_workers
    rows_per_chunk = b_per_w // _SC_CHUNKS
    mesh = plsc.VectorSubcoreMesh(core_axis_name="c", subcore_axis_name="s")

    @functools.partial(
        pl.kernel,
        mesh=mesh,
        out_type=jax.ShapeDtypeStruct((b, d), jnp.float32),
        scratch_types=[
            pltpu.VMEM((b_per_w,), jnp.int32),
        ]
        + [pltpu.VMEM((rows_per_chunk, d), jnp.float32)] * _SC_CHUNKS
        + [pltpu.SemaphoreType.DMA] * (2 * _SC_CHUNKS),
    )
    def gather_k(table_hbm, idx_hbm, out_hbm, idx_v, *bufs_and_sems):
        bufs = bufs_and_sems[:_SC_CHUNKS]
        gsems = bufs_and_sems[_SC_CHUNKS : 2 * _SC_CHUNKS]
        osems = bufs_and_sems[2 * _SC_CHUNKS :]
        wid = lax.axis_index("s") * info.num_cores + lax.axis_index("c")
        base = wid * b_per_w
        pltpu.sync_copy(idx_hbm.at[pl.ds(base, b_per_w)], idx_v)
        gathers = [
            pltpu.async_copy(
                table_hbm.at[idx_v.at[pl.ds(c * rows_per_chunk, rows_per_chunk)]],
                bufs[c],
                gsems[c],
            )
            for c in range(_SC_CHUNKS)
        ]
        scatters = []
        for c in range(_SC_CHUNKS):
            gathers[c].wait()
            scatters.append(
                pltpu.async_copy(
                    bufs[c],
                    out_hbm.at[pl.ds(base + c * rows_per_chunk, rows_per_chunk)],
                    osems[c],
                )
            )
        for s in scatters:
            s.wait()

    return gather_k(table, idx_flat)


_BATCHES_PER_STEP = 2


def _tc_body(tok_ref, vc_ref, out_ref):
    for j in range(_BATCHES_PER_STEP):
        t = tok_ref[j]
        vc = vc_ref[j]
        k = vc.shape[0]
        sims = lax.dot_general(
            vc, t, (((1,), (1,)), ((), ())), preferred_element_type=jnp.float32
        )
        s2 = jnp.concatenate([sims, -sims], axis=0)
        col = lax.broadcasted_iota(jnp.int32, s2.shape, 1)
        s2 = jnp.where(col == 0, -jnp.inf, s2)
        m = jnp.max(s2, axis=-1, keepdims=True)
        e = jnp.exp(s2 - m)
        p = e / jnp.sum(e, axis=-1, keepdims=True)
        w = lax.dot_general(
            p, t, (((1,), (0,)), ((), ())), preferred_element_type=jnp.float32
        )
        nrm = jnp.sqrt(jnp.sum(w * w, axis=-1, keepdims=True))
        w = w / jnp.maximum(nrm, 1e-12)
        out_ref[0, j] = w[:k]
        out_ref[1, j] = w[k:]


def kernel(tokens, topk_indices, visual_concepts):
    b, n1, d = tokens.shape
    k = topk_indices.shape[1]
    idx_flat = topk_indices.reshape(-1).astype(jnp.int32)
    vc = _sc_gather(visual_concepts, idx_flat).reshape(b, k, d)
    g = _BATCHES_PER_STEP
    out = pl.pallas_call(
        _tc_body,
        grid=(b // g,),
        in_specs=[
            pl.BlockSpec((g, n1, d), lambda i: (i, 0, 0)),
            pl.BlockSpec((g, k, d), lambda i: (i, 0, 0)),
        ],
        out_specs=pl.BlockSpec((2, g, k, d), lambda i: (0, i, 0, 0)),
        out_shape=jax.ShapeDtypeStruct((2, b, k, d), jnp.float32),
        compiler_params=pltpu.CompilerParams(
            dimension_semantics=("parallel",),
        ),
    )(tokens, vc)
    return out


# speedup vs baseline: 1.5374x; 1.0685x over previous
import functools

import jax
import jax.numpy as jnp
from jax import lax
from jax.experimental import pallas as pl
from jax.experimental.pallas import tpu as pltpu
from jax.experimental.pallas import tpu_sc as plsc


_SC_CHUNKS = 4


def _sc_gather(table, idx_flat):
    info = plsc.get_sparse_core_info()
    num_workers = info.num_cores * info.num_subcores
    b = idx_flat.shape[0]
    d = table.shape[1]
    b_per_w = b // num_workers
    rows_per_chunk = b_per_w // _SC_CHUNKS
    mesh = plsc.VectorSubcoreMesh(core_axis_name="c", subcore_axis_name="s")

    @functools.partial(
        pl.kernel,
        mesh=mesh,
        out_type=jax.ShapeDtypeStruct((b, d), jnp.float32),
        scratch_types=[
            pltpu.VMEM((b_per_w,), jnp.int32),
        ]
        + [pltpu.VMEM((rows_per_chunk, d), jnp.float32)] * _SC_CHUNKS
        + [pltpu.SemaphoreType.DMA] * (2 * _SC_CHUNKS),
    )
    def gather_k(table_hbm, idx_hbm, out_hbm, idx_v, *bufs_and_sems):
        bufs = bufs_and_sems[:_SC_CHUNKS]
        gsems = bufs_and_sems[_SC_CHUNKS : 2 * _SC_CHUNKS]
        osems = bufs_and_sems[2 * _SC_CHUNKS :]
        wid = lax.axis_index("s") * info.num_cores + lax.axis_index("c")
        base = wid * b_per_w
        pltpu.sync_copy(idx_hbm.at[pl.ds(base, b_per_w)], idx_v)
        gathers = [
            pltpu.async_copy(
                table_hbm.at[idx_v.at[pl.ds(c * rows_per_chunk, rows_per_chunk)]],
                bufs[c],
                gsems[c],
            )
            for c in range(_SC_CHUNKS)
        ]
        scatters = []
        for c in range(_SC_CHUNKS):
            gathers[c].wait()
            scatters.append(
                pltpu.async_copy(
                    bufs[c],
                    out_hbm.at[pl.ds(base + c * rows_per_chunk, rows_per_chunk)],
                    osems[c],
                )
            )
        for s in scatters:
            s.wait()

    return gather_k(table, idx_flat)


_BATCHES_PER_STEP = 4


def _tc_body(tok_ref, vc_ref, out_ref):
    for j in range(_BATCHES_PER_STEP):
        t = tok_ref[j]
        vc = vc_ref[j]
        k = vc.shape[0]
        sims = lax.dot_general(
            vc, t, (((1,), (1,)), ((), ())), preferred_element_type=jnp.float32
        )
        s2 = jnp.concatenate([sims, -sims], axis=0)
        col = lax.broadcasted_iota(jnp.int32, s2.shape, 1)
        s2 = jnp.where(col == 0, -jnp.inf, s2)
        m = jnp.max(s2, axis=-1, keepdims=True)
        e = jnp.exp(s2 - m)
        p = e / jnp.sum(e, axis=-1, keepdims=True)
        w = lax.dot_general(
            p, t, (((1,), (0,)), ((), ())), preferred_element_type=jnp.float32
        )
        nrm = jnp.sqrt(jnp.sum(w * w, axis=-1, keepdims=True))
        w = w / jnp.maximum(nrm, 1e-12)
        out_ref[0, j] = w[:k]
        out_ref[1, j] = w[k:]


def kernel(tokens, topk_indices, visual_concepts):
    b, n1, d = tokens.shape
    k = topk_indices.shape[1]
    idx_flat = topk_indices.reshape(-1).astype(jnp.int32)
    vc = _sc_gather(visual_concepts, idx_flat).reshape(b, k, d)
    g = _BATCHES_PER_STEP
    out = pl.pallas_call(
        _tc_body,
        grid=(b // g,),
        in_specs=[
            pl.BlockSpec((g, n1, d), lambda i: (i, 0, 0)),
            pl.BlockSpec((g, k, d), lambda i: (i, 0, 0)),
        ],
        out_specs=pl.BlockSpec((2, g, k, d), lambda i: (0, i, 0, 0)),
        out_shape=jax.ShapeDtypeStruct((2, b, k, d), jnp.float32),
        compiler_params=pltpu.CompilerParams(
            dimension_semantics=("parallel",),
        ),
    )(tokens, vc)
    return out
